# baseline (speedup 1.0000x reference)
import jax
import jax.numpy as jnp
from jax.experimental import pallas as pl

SIDE = 32
N = 1024
D = 96
NC = N - 2 * SIDE


def _patch_attention_kernel(x_ref, out_ref):
    xb = x_ref[0]
    s = jnp.concatenate([xb[:SIDE], xb[N - SIDE:]], axis=0)
    c = xb[SIDE:N - SIDE]

    snorm = jnp.sqrt(jnp.sum(s * s, axis=1, keepdims=True))
    s_n = s / jnp.maximum(snorm, 1e-12)
    cnorm = jnp.sqrt(jnp.sum(c * c, axis=1, keepdims=True))
    c_n = c / jnp.maximum(cnorm, 1e-12)

    scaled = jax.lax.dot_general(
        s_n, c_n, (((1,), (1,)), ((), ())),
        preferred_element_type=jnp.float32)

    iota = jax.lax.broadcasted_iota(jnp.int32, (2 * SIDE, NC), 1)
    mx = jnp.max(scaled, axis=1, keepdims=True)
    idx = jnp.min(jnp.where(scaled == mx, iota, NC), axis=1, keepdims=True)
    one_hot = (iota == idx).astype(jnp.float32)

    edges = jax.lax.dot_general(
        one_hot, c, (((1,), (0,)), ((), ())),
        preferred_element_type=jnp.float32)

    out_ref[0, :SIDE] = edges[:SIDE]
    out_ref[0, SIDE:N - SIDE] = c
    out_ref[0, N - SIDE:] = edges[SIDE:]


def kernel(x):
    B = x.shape[0]
    return pl.pallas_call(
        _patch_attention_kernel,
        grid=(B,),
        in_specs=[pl.BlockSpec((1, N, D), lambda b: (b, 0, 0))],
        out_specs=pl.BlockSpec((1, N, D), lambda b: (b, 0, 0)),
        out_shape=jax.ShapeDtypeStruct((B, N, D), x.dtype),
    )(x)
